# Initial kernel scaffold; baseline (speedup 1.0000x reference)
import jax
import jax.numpy as jnp
from jax.experimental import pallas as pl


def kernel(x, edge_attr, edge_index, eW1, eb1, eW2, eb2, nW1, nb1, nW2, nb2):
    raise NotImplementedError("write your pallas kernel here")


# speedup vs baseline: 3.2284x; 3.2284x over previous
import functools

import jax
import jax.numpy as jnp
from jax import lax
from jax.experimental import pallas as pl
from jax.experimental.pallas import tpu as pltpu
from jax.experimental.pallas import tpu_sc as plsc

N = 10000
E = 320000
D = 128
NPAD = 10240
W = 128
NWIN = E // W
NC = 2
NS = 16
NWK = NC * NS
ZR = NPAD // NS

_mesh = plsc.VectorSubcoreMesh(core_axis_name="c", subcore_axis_name="s")


def _pre(x, w1a, w1b, na, eb1, nb1):
    def body(x_r, wa_r, wb_r, na_r, eb1_r, nb1_r, xa_r, xb_r, xn_r):
        xv = x_r[...]
        xa_r[...] = jnp.dot(xv, wa_r[...], preferred_element_type=jnp.float32) + eb1_r[...]
        xb_r[...] = jnp.dot(xv, wb_r[...], preferred_element_type=jnp.float32)
        xn_r[...] = jnp.dot(xv, na_r[...], preferred_element_type=jnp.float32) + nb1_r[...]

    NB = 2000
    blk = pl.BlockSpec((NB, D), lambda i: (i, 0))
    wblk = pl.BlockSpec((D, D), lambda i: (0, 0))
    bblk = pl.BlockSpec((1, D), lambda i: (0, 0))
    out = jax.ShapeDtypeStruct((N, D), jnp.float32)
    return pl.pallas_call(
        body,
        grid=(N // NB,),
        in_specs=[blk, wblk, wblk, wblk, bblk, bblk],
        out_specs=(blk, blk, blk),
        out_shape=(out, out, out),
    )(x, w1a, w1b, na, eb1, nb1)


def _gather(xa, xb, row, col):
    @functools.partial(
        pl.kernel,
        out_type=(jax.ShapeDtypeStruct((E, D), jnp.float32),
                  jax.ShapeDtypeStruct((E, D), jnp.float32)),
        mesh=_mesh,
        scratch_types=[
            pltpu.VMEM((W,), jnp.int32),
            pltpu.VMEM((W,), jnp.int32),
            pltpu.VMEM((W, D), jnp.float32),
            pltpu.VMEM((W, D), jnp.float32),
            pltpu.SemaphoreType.DMA,
            pltpu.SemaphoreType.DMA,
        ],
    )
    def k(xa_hbm, xb_hbm, row_hbm, col_hbm, g1_hbm, g2_hbm,
          ridx, cidx, r1, r2, s1, s2):
        cid = lax.axis_index("c")
        sid = lax.axis_index("s")
        wid = sid * NC + cid

        @pl.loop(0, NWIN // NWK + 1)
        def _(j):
            w = wid + j * NWK

            @pl.when(w < NWIN)
            def _():
                base = w * W
                pltpu.sync_copy(row_hbm.at[pl.ds(base, W)], ridx)
                pltpu.sync_copy(col_hbm.at[pl.ds(base, W)], cidx)
                c1 = pltpu.async_copy(xa_hbm.at[ridx], r1, s1)
                c2 = pltpu.async_copy(xb_hbm.at[cidx], r2, s2)
                c1.wait()
                c2.wait()
                pltpu.sync_copy(r1, g1_hbm.at[pl.ds(base, W)])
                pltpu.sync_copy(r2, g2_hbm.at[pl.ds(base, W)])

    return k(xa, xb, row, col)


def _edge(g1, g2, ea, w1c, w2, eb2):
    EB = 1280

    def body(g1_r, g2_r, ea_r, w1_r, w2_r, b2_r, out_r):
        eav = ea_r[...]
        c = jnp.dot(eav, w1_r[...], preferred_element_type=jnp.float32)
        h = jnp.maximum(g1_r[...] + g2_r[...] + c, 0.0)
        out_r[...] = eav + jnp.dot(h, w2_r[...], preferred_element_type=jnp.float32) + b2_r[...]

    blk = pl.BlockSpec((EB, D), lambda i: (i, 0))
    wblk = pl.BlockSpec((D, D), lambda i: (0, 0))
    bblk = pl.BlockSpec((1, D), lambda i: (0, 0))
    return pl.pallas_call(
        body,
        grid=(E // EB,),
        in_specs=[blk, blk, blk, wblk, wblk, bblk],
        out_specs=blk,
        out_shape=jax.ShapeDtypeStruct((E, D), jnp.float32),
    )(g1, g2, ea, w1c, w2, eb2)


def _scatter(eout, col, zrows):
    @functools.partial(
        pl.kernel,
        out_type=jax.ShapeDtypeStruct((NC, NPAD, D), jnp.float32),
        mesh=_mesh,
        scratch_types=[
            pltpu.VMEM((W,), jnp.int32),
            pltpu.VMEM((W, D), jnp.float32),
            pltpu.VMEM_SHARED((NPAD, D), jnp.float32),
            pltpu.SemaphoreType.DMA,
        ],
    )
    def k(e_hbm, col_hbm, z_hbm, out_hbm, cidx, ebuf, acc, sem):
        cid = lax.axis_index("c")
        sid = lax.axis_index("s")
        wid = sid * NC + cid

        pltpu.sync_copy(z_hbm, acc.at[pl.ds(sid * ZR, ZR)])
        plsc.subcore_barrier()

        @pl.loop(0, NWIN // NWK + 1)
        def _(j):
            w = wid + j * NWK

            @pl.when(w < NWIN)
            def _():
                base = w * W
                pltpu.sync_copy(col_hbm.at[pl.ds(base, W)], cidx)
                pltpu.sync_copy(e_hbm.at[pl.ds(base, W)], ebuf)
                pltpu.sync_copy(ebuf, acc.at[cidx], add=True)

        plsc.subcore_barrier()
        pltpu.sync_copy(acc.at[pl.ds(sid * ZR, ZR)],
                        out_hbm.at[cid, pl.ds(sid * ZR, ZR)])

    return k(eout, col, zrows)


def _node(x, xn, partials, nbw, nw2, nb2):
    NB = 2000

    def body(x_r, xn_r, p_r, nb_r, w2_r, b2_r, out_r):
        agg = p_r[0] + p_r[1]
        h2 = jnp.maximum(xn_r[...] + jnp.dot(agg, nb_r[...], preferred_element_type=jnp.float32), 0.0)
        out_r[...] = x_r[...] + jnp.dot(h2, w2_r[...], preferred_element_type=jnp.float32) + b2_r[...]

    blk = pl.BlockSpec((NB, D), lambda i: (i, 0))
    pblk = pl.BlockSpec((NC, NB, D), lambda i: (0, i, 0))
    wblk = pl.BlockSpec((D, D), lambda i: (0, 0))
    bblk = pl.BlockSpec((1, D), lambda i: (0, 0))
    return pl.pallas_call(
        body,
        grid=(N // NB,),
        in_specs=[blk, blk, pblk, wblk, wblk, bblk],
        out_specs=blk,
        out_shape=jax.ShapeDtypeStruct((N, D), jnp.float32),
    )(x, xn, partials, nbw, nw2, nb2)


def kernel(x, edge_attr, edge_index, eW1, eb1, eW2, eb2, nW1, nb1, nW2, nb2):
    row = edge_index[0].astype(jnp.int32)
    col = edge_index[1].astype(jnp.int32)
    w1a, w1b, w1c = eW1[:D], eW1[D:2 * D], eW1[2 * D:]
    na, nbw = nW1[:D], nW1[D:]
    eb1r = eb1.reshape(1, D)
    eb2r = eb2.reshape(1, D)
    nb1r = nb1.reshape(1, D)
    nb2r = nb2.reshape(1, D)

    xa, xb, xn = _pre(x, w1a, w1b, na, eb1r, nb1r)
    g1, g2 = _gather(xa, xb, row, col)
    eout = _edge(g1, g2, edge_attr, w1c, eW2, eb2r)
    zrows = jnp.zeros((ZR, D), jnp.float32)
    partials = _scatter(eout, col, zrows)
    x_new = _node(x, xn, partials, nbw, nW2, nb2r)
    return (x_new, eout)


# speedup vs baseline: 4.1077x; 1.2724x over previous
import functools

import jax
import jax.numpy as jnp
from jax import lax
from jax.experimental import pallas as pl
from jax.experimental.pallas import tpu as pltpu
from jax.experimental.pallas import tpu_sc as plsc

N = 10000
E = 320000
D = 128
NPAD = 10240
W = 128
NWIN = E // W
NC = 2
NS = 16
NWK = NC * NS
ZR = NPAD // NS

_mesh = plsc.VectorSubcoreMesh(core_axis_name="c", subcore_axis_name="s")


def _pre(x, w1a, w1b, na, eb1, nb1):
    def body(x_r, wa_r, wb_r, na_r, eb1_r, nb1_r, xa_r, xb_r, xn_r):
        xv = x_r[...]
        xa_r[...] = jnp.dot(xv, wa_r[...], preferred_element_type=jnp.float32) + eb1_r[...]
        xb_r[...] = jnp.dot(xv, wb_r[...], preferred_element_type=jnp.float32)
        xn_r[...] = jnp.dot(xv, na_r[...], preferred_element_type=jnp.float32) + nb1_r[...]

    NB = 2000
    blk = pl.BlockSpec((NB, D), lambda i: (i, 0))
    wblk = pl.BlockSpec((D, D), lambda i: (0, 0))
    bblk = pl.BlockSpec((1, D), lambda i: (0, 0))
    out = jax.ShapeDtypeStruct((N, D), jnp.float32)
    return pl.pallas_call(
        body,
        grid=(N // NB,),
        in_specs=[blk, wblk, wblk, wblk, bblk, bblk],
        out_specs=(blk, blk, blk),
        out_shape=(out, out, out),
    )(x, w1a, w1b, na, eb1, nb1)


NJMAX = NWIN // NWK + 1


def _wrange(wid):
    w0 = wid * (NWIN // NWK) + jnp.minimum(wid, NWIN % NWK)
    cnt = NWIN // NWK + (wid < NWIN % NWK).astype(jnp.int32)
    return w0, cnt


def _gather(xa, xb, row2, col2):
    @functools.partial(
        pl.kernel,
        out_type=(jax.ShapeDtypeStruct((E, D), jnp.float32),
                  jax.ShapeDtypeStruct((E, D), jnp.float32)),
        mesh=_mesh,
        scratch_types=[
            pltpu.VMEM((NJMAX, W), jnp.int32),
            pltpu.VMEM((NJMAX, W), jnp.int32),
            pltpu.VMEM((W, D), jnp.float32),
            pltpu.VMEM((W, D), jnp.float32),
            pltpu.VMEM((W, D), jnp.float32),
            pltpu.VMEM((W, D), jnp.float32),
            pltpu.SemaphoreType.DMA,
            pltpu.SemaphoreType.DMA,
        ],
    )
    def k(xa_hbm, xb_hbm, row_hbm, col_hbm, g1_hbm, g2_hbm,
          ridx, cidx, r1a, r2a, r1b, r2b, sa, sb):
        cid = lax.axis_index("c")
        sid = lax.axis_index("s")
        wid = sid * NC + cid
        w0, cnt = _wrange(wid)

        pltpu.sync_copy(row_hbm.at[wid], ridx)
        pltpu.sync_copy(col_hbm.at[wid], cidx)

        pltpu.async_copy(xa_hbm.at[ridx.at[0]], r1a, sa)
        pltpu.async_copy(xb_hbm.at[cidx.at[0]], r2a, sa)

        def slot(k_, my1, my2, mysem, ot1, ot2, otsem):
            @pl.when(k_ < cnt)
            def _():
                pltpu.make_async_copy(xa_hbm.at[ridx.at[0]], my1, mysem).wait()
                pltpu.make_async_copy(xb_hbm.at[cidx.at[0]], my2, mysem).wait()

                @pl.when(k_ + 1 < cnt)
                def _():
                    pltpu.async_copy(xa_hbm.at[ridx.at[k_ + 1]], ot1, otsem)
                    pltpu.async_copy(xb_hbm.at[cidx.at[k_ + 1]], ot2, otsem)

                base = (w0 + k_) * W
                pltpu.sync_copy(my1, g1_hbm.at[pl.ds(base, W)])
                pltpu.sync_copy(my2, g2_hbm.at[pl.ds(base, W)])

        @pl.loop(0, NJMAX + 1, step=2)
        def _(k_):
            slot(k_, r1a, r2a, sa, r1b, r2b, sb)
            slot(k_ + 1, r1b, r2b, sb, r1a, r2a, sa)

    return k(xa, xb, row2, col2)


def _edge(g1, g2, ea, w1c, w2, eb2):
    EB = 1280

    def body(g1_r, g2_r, ea_r, w1_r, w2_r, b2_r, out_r):
        eav = ea_r[...]
        c = jnp.dot(eav, w1_r[...], preferred_element_type=jnp.float32)
        h = jnp.maximum(g1_r[...] + g2_r[...] + c, 0.0)
        out_r[...] = eav + jnp.dot(h, w2_r[...], preferred_element_type=jnp.float32) + b2_r[...]

    blk = pl.BlockSpec((EB, D), lambda i: (i, 0))
    wblk = pl.BlockSpec((D, D), lambda i: (0, 0))
    bblk = pl.BlockSpec((1, D), lambda i: (0, 0))
    return pl.pallas_call(
        body,
        grid=(E // EB,),
        in_specs=[blk, blk, blk, wblk, wblk, bblk],
        out_specs=blk,
        out_shape=jax.ShapeDtypeStruct((E, D), jnp.float32),
    )(g1, g2, ea, w1c, w2, eb2)


def _scatter(eout, col2, zrows):
    @functools.partial(
        pl.kernel,
        out_type=jax.ShapeDtypeStruct((NC, NPAD, D), jnp.float32),
        mesh=_mesh,
        scratch_types=[
            pltpu.VMEM((NJMAX, W), jnp.int32),
            pltpu.VMEM((W, D), jnp.float32),
            pltpu.VMEM((W, D), jnp.float32),
            pltpu.VMEM_SHARED((NPAD, D), jnp.float32),
            pltpu.SemaphoreType.DMA,
            pltpu.SemaphoreType.DMA,
        ],
    )
    def k(e_hbm, col_hbm, z_hbm, out_hbm, cidx, ebufa, ebufb, acc, sa, sb):
        cid = lax.axis_index("c")
        sid = lax.axis_index("s")
        wid = sid * NC + cid
        w0, cnt = _wrange(wid)

        pltpu.sync_copy(z_hbm, acc.at[pl.ds(sid * ZR, ZR)])
        pltpu.sync_copy(col_hbm.at[wid], cidx)
        plsc.subcore_barrier()

        pltpu.async_copy(e_hbm.at[pl.ds(w0 * W, W)], ebufa, sa)

        def slot(k_, mybuf, mysem, otbuf, otsem):
            @pl.when(k_ < cnt)
            def _():
                pltpu.make_async_copy(e_hbm.at[pl.ds(0, W)], mybuf, mysem).wait()

                @pl.when(k_ + 1 < cnt)
                def _():
                    pltpu.async_copy(e_hbm.at[pl.ds((w0 + k_ + 1) * W, W)],
                                     otbuf, otsem)

                pltpu.sync_copy(mybuf, acc.at[cidx.at[k_]], add=True)

        @pl.loop(0, NJMAX + 1, step=2)
        def _(k_):
            slot(k_, ebufa, sa, ebufb, sb)
            slot(k_ + 1, ebufb, sb, ebufa, sa)

        plsc.subcore_barrier()
        pltpu.sync_copy(acc.at[pl.ds(sid * ZR, ZR)],
                        out_hbm.at[cid, pl.ds(sid * ZR, ZR)])

    return k(eout, col2, zrows)


def _node(x, xn, partials, nbw, nw2, nb2):
    NB = 2000

    def body(x_r, xn_r, p_r, nb_r, w2_r, b2_r, out_r):
        agg = p_r[0] + p_r[1]
        h2 = jnp.maximum(xn_r[...] + jnp.dot(agg, nb_r[...], preferred_element_type=jnp.float32), 0.0)
        out_r[...] = x_r[...] + jnp.dot(h2, w2_r[...], preferred_element_type=jnp.float32) + b2_r[...]

    blk = pl.BlockSpec((NB, D), lambda i: (i, 0))
    pblk = pl.BlockSpec((NC, NB, D), lambda i: (0, i, 0))
    wblk = pl.BlockSpec((D, D), lambda i: (0, 0))
    bblk = pl.BlockSpec((1, D), lambda i: (0, 0))
    return pl.pallas_call(
        body,
        grid=(N // NB,),
        in_specs=[blk, blk, pblk, wblk, wblk, bblk],
        out_specs=blk,
        out_shape=jax.ShapeDtypeStruct((N, D), jnp.float32),
    )(x, xn, partials, nbw, nw2, nb2)


def kernel(x, edge_attr, edge_index, eW1, eb1, eW2, eb2, nW1, nb1, nW2, nb2):
    row = edge_index[0].astype(jnp.int32)
    col = edge_index[1].astype(jnp.int32)
    w1a, w1b, w1c = eW1[:D], eW1[D:2 * D], eW1[2 * D:]
    na, nbw = nW1[:D], nW1[D:]
    eb1r = eb1.reshape(1, D)
    eb2r = eb2.reshape(1, D)
    nb1r = nb1.reshape(1, D)
    nb2r = nb2.reshape(1, D)

    row2 = jnp.pad(row, (0, W)).reshape(NWIN + 1, W)
    col2 = jnp.pad(col, (0, W)).reshape(NWIN + 1, W)
    w0s = [t * (NWIN // NWK) + min(t, NWIN % NWK) for t in range(NWK)]
    row3 = jnp.stack([lax.slice(row2, (s, 0), (s + NJMAX, W)) for s in w0s])
    col3 = jnp.stack([lax.slice(col2, (s, 0), (s + NJMAX, W)) for s in w0s])

    xa, xb, xn = _pre(x, w1a, w1b, na, eb1r, nb1r)
    g1, g2 = _gather(xa, xb, row3, col3)
    eout = _edge(g1, g2, edge_attr, w1c, eW2, eb2r)
    zrows = jnp.zeros((ZR, D), jnp.float32)
    partials = _scatter(eout, col3, zrows)
    x_new = _node(x, xn, partials, nbw, nW2, nb2r)
    return (x_new, eout)


# speedup vs baseline: 4.5458x; 1.1067x over previous
import functools

import jax
import jax.numpy as jnp
from jax import lax
from jax.experimental import pallas as pl
from jax.experimental.pallas import tpu as pltpu
from jax.experimental.pallas import tpu_sc as plsc

N = 10000
E = 320000
D = 128
NPAD = 10240
W = 128
NWIN = E // W
NC = 2
NS = 16
NWK = NC * NS
ZR = NPAD // NS

_mesh = plsc.VectorSubcoreMesh(core_axis_name="c", subcore_axis_name="s")


def _pre(x, w1a, w1b, na, eb1, nb1):
    def body(x_r, wa_r, wb_r, na_r, eb1_r, nb1_r, xa_r, xb_r, xn_r):
        xv = x_r[...]
        xa_r[...] = jnp.dot(xv, wa_r[...], preferred_element_type=jnp.float32) + eb1_r[...]
        xb_r[...] = jnp.dot(xv, wb_r[...], preferred_element_type=jnp.float32)
        xn_r[...] = jnp.dot(xv, na_r[...], preferred_element_type=jnp.float32) + nb1_r[...]

    NB = 2000
    blk = pl.BlockSpec((NB, D), lambda i: (i, 0))
    wblk = pl.BlockSpec((D, D), lambda i: (0, 0))
    bblk = pl.BlockSpec((1, D), lambda i: (0, 0))
    outf = jax.ShapeDtypeStruct((N, D), jnp.float32)
    return pl.pallas_call(
        body,
        grid=(N // NB,),
        in_specs=[blk, wblk, wblk, wblk, bblk, bblk],
        out_specs=(blk, blk, blk),
        out_shape=(outf, outf, outf),
    )(x, w1a, w1b, na, eb1, nb1)


NJMAX = NWIN // NWK + 1


def _wrange(wid):
    w0 = wid * (NWIN // NWK) + jnp.minimum(wid, NWIN % NWK)
    cnt = NWIN // NWK + (wid < NWIN % NWK).astype(jnp.int32)
    return w0, cnt


def _gather(xa, xb, row2, col2, idmat):
    @functools.partial(
        pl.kernel,
        out_type=jax.ShapeDtypeStruct((E, D), jnp.float32),
        mesh=_mesh,
        scratch_types=[
            pltpu.VMEM((NJMAX, W), jnp.int32),
            pltpu.VMEM((NJMAX, W), jnp.int32),
            pltpu.VMEM((1, W), jnp.int32),
            pltpu.VMEM((W, D), jnp.float32),
            pltpu.VMEM((W, D), jnp.float32),
            pltpu.VMEM((W, D), jnp.float32),
            pltpu.VMEM((W, D), jnp.float32),
            pltpu.VMEM_SHARED((NS * W, D), jnp.float32),
            pltpu.VMEM_SHARED((NS * W, D), jnp.float32),
            pltpu.SemaphoreType.DMA,
            pltpu.SemaphoreType.DMA,
        ],
    )
    def k(xa_hbm, xb_hbm, row_hbm, col_hbm, idm_hbm, g_hbm,
          ridx, cidx, ida, r1a, r2a, r1b, r2b, spa, spb, sa, sb):
        cid = lax.axis_index("c")
        sid = lax.axis_index("s")
        wid = sid * NC + cid
        w0, cnt = _wrange(wid)
        my0 = sid * W

        pltpu.sync_copy(row_hbm.at[wid], ridx)
        pltpu.sync_copy(col_hbm.at[wid], cidx)
        pltpu.sync_copy(idm_hbm.at[sid], ida)

        pltpu.async_copy(xa_hbm.at[ridx.at[0]], r1a, sa)
        pltpu.async_copy(xb_hbm.at[cidx.at[0]], r2a, sa)

        def slot(k_, my1, my2, mysem, ot1, ot2, otsem, mysp):
            @pl.when(k_ < cnt)
            def _():
                pltpu.make_async_copy(xa_hbm.at[ridx.at[0]], my1, mysem).wait()
                pltpu.make_async_copy(xb_hbm.at[cidx.at[0]], my2, mysem).wait()

                @pl.when(k_ + 1 < cnt)
                def _():
                    pltpu.async_copy(xa_hbm.at[ridx.at[k_ + 1]], ot1, otsem)
                    pltpu.async_copy(xb_hbm.at[cidx.at[k_ + 1]], ot2, otsem)

                pltpu.sync_copy(my1, mysp.at[pl.ds(my0, W)])
                pltpu.sync_copy(my2, mysp.at[ida.at[0]], add=True)
                pltpu.sync_copy(mysp.at[pl.ds(my0, W)],
                                g_hbm.at[pl.ds((w0 + k_) * W, W)])

        @pl.loop(0, NJMAX + 1, step=2)
        def _(k_):
            slot(k_, r1a, r2a, sa, r1b, r2b, sb, spa)
            slot(k_ + 1, r1b, r2b, sb, r1a, r2a, sa, spb)

    return k(xa, xb, row2, col2, idmat)


def _edge(g, ea, w1c, w2, eb2):
    EB = 1280

    def body(g_r, ea_r, w1_r, w2_r, b2_r, out_r):
        eav = ea_r[...]
        c = jnp.dot(eav.astype(jnp.bfloat16), w1_r[...], preferred_element_type=jnp.float32)
        h = jnp.maximum(g_r[...] + c, 0.0)
        out_r[...] = (eav + jnp.dot(h.astype(jnp.bfloat16), w2_r[...],
                                    preferred_element_type=jnp.float32) + b2_r[...])

    blk = pl.BlockSpec((EB, D), lambda i: (i, 0))
    wblk = pl.BlockSpec((D, D), lambda i: (0, 0))
    bblk = pl.BlockSpec((1, D), lambda i: (0, 0))
    return pl.pallas_call(
        body,
        grid=(E // EB,),
        in_specs=[blk, blk, wblk, wblk, bblk],
        out_specs=blk,
        out_shape=jax.ShapeDtypeStruct((E, D), jnp.float32),
    )(g, ea, w1c, w2, eb2)


def _scatter(eout, col2, zrows):
    @functools.partial(
        pl.kernel,
        out_type=jax.ShapeDtypeStruct((NC, NPAD, D), jnp.float32),
        mesh=_mesh,
        scratch_types=[
            pltpu.VMEM((NJMAX, W), jnp.int32),
            pltpu.VMEM((W, D), jnp.float32),
            pltpu.VMEM((W, D), jnp.float32),
            pltpu.VMEM_SHARED((NPAD, D), jnp.float32),
            pltpu.SemaphoreType.DMA,
            pltpu.SemaphoreType.DMA,
        ],
    )
    def k(e_hbm, col_hbm, z_hbm, out_hbm, cidx, ebufa, ebufb, acc, sa, sb):
        cid = lax.axis_index("c")
        sid = lax.axis_index("s")
        wid = sid * NC + cid
        w0, cnt = _wrange(wid)

        pltpu.sync_copy(z_hbm, acc.at[pl.ds(sid * ZR, ZR)])
        pltpu.sync_copy(col_hbm.at[wid], cidx)
        plsc.subcore_barrier()

        pltpu.async_copy(e_hbm.at[pl.ds(w0 * W, W)], ebufa, sa)

        def slot(k_, mybuf, mysem, otbuf, otsem):
            @pl.when(k_ < cnt)
            def _():
                pltpu.make_async_copy(e_hbm.at[pl.ds(0, W)], mybuf, mysem).wait()

                @pl.when(k_ + 1 < cnt)
                def _():
                    pltpu.async_copy(e_hbm.at[pl.ds((w0 + k_ + 1) * W, W)],
                                     otbuf, otsem)

                pltpu.sync_copy(mybuf, acc.at[cidx.at[k_]], add=True)

        @pl.loop(0, NJMAX + 1, step=2)
        def _(k_):
            slot(k_, ebufa, sa, ebufb, sb)
            slot(k_ + 1, ebufb, sb, ebufa, sa)

        plsc.subcore_barrier()
        pltpu.sync_copy(acc.at[pl.ds(sid * ZR, ZR)],
                        out_hbm.at[cid, pl.ds(sid * ZR, ZR)])

    return k(eout, col2, zrows)


def _node(x, xn, partials, nbw, nw2, nb2):
    NB = 2000

    def body(x_r, xn_r, p_r, nb_r, w2_r, b2_r, out_r):
        agg = p_r[0] + p_r[1]
        h2 = jnp.maximum(xn_r[...] + jnp.dot(agg, nb_r[...], preferred_element_type=jnp.float32), 0.0)
        out_r[...] = x_r[...] + jnp.dot(h2, w2_r[...], preferred_element_type=jnp.float32) + b2_r[...]

    blk = pl.BlockSpec((NB, D), lambda i: (i, 0))
    pblk = pl.BlockSpec((NC, NB, D), lambda i: (0, i, 0))
    wblk = pl.BlockSpec((D, D), lambda i: (0, 0))
    bblk = pl.BlockSpec((1, D), lambda i: (0, 0))
    return pl.pallas_call(
        body,
        grid=(N // NB,),
        in_specs=[blk, blk, pblk, wblk, wblk, bblk],
        out_specs=blk,
        out_shape=jax.ShapeDtypeStruct((N, D), jnp.float32),
    )(x, xn, partials, nbw, nw2, nb2)


def kernel(x, edge_attr, edge_index, eW1, eb1, eW2, eb2, nW1, nb1, nW2, nb2):
    row = edge_index[0].astype(jnp.int32)
    col = edge_index[1].astype(jnp.int32)
    w1a, w1b, w1c = eW1[:D], eW1[D:2 * D], eW1[2 * D:]
    na, nbw = nW1[:D], nW1[D:]
    eb1r = eb1.reshape(1, D)
    eb2r = eb2.reshape(1, D)
    nb1r = nb1.reshape(1, D)
    nb2r = nb2.reshape(1, D)

    row2 = jnp.pad(row, (0, W)).reshape(NWIN + 1, W)
    col2 = jnp.pad(col, (0, W)).reshape(NWIN + 1, W)
    w0s = [t * (NWIN // NWK) + min(t, NWIN % NWK) for t in range(NWK)]
    row3 = jnp.stack([lax.slice(row2, (s, 0), (s + NJMAX, W)) for s in w0s])
    col3 = jnp.stack([lax.slice(col2, (s, 0), (s + NJMAX, W)) for s in w0s])

    xa, xb, xn = _pre(x, w1a, w1b, na, eb1r, nb1r)
    idmat = (jnp.arange(NS, dtype=jnp.int32)[:, None, None] * W
             + jnp.arange(W, dtype=jnp.int32)[None, None, :])
    g = _gather(xa, xb, row3, col3, idmat)
    eout = _edge(g, edge_attr, w1c.astype(jnp.bfloat16),
                 eW2.astype(jnp.bfloat16), eb2r)
    zrows = jnp.zeros((ZR, D), jnp.float32)
    partials = _scatter(eout, col3, zrows)
    x_new = _node(x, xn, partials, nbw, nW2, nb2r)
    return (x_new, eout)
